# baseline (speedup 1.0000x reference)
import jax
import jax.numpy as jnp
from jax import lax
from jax.experimental import pallas as pl
from jax.experimental.pallas import tpu as pltpu

SCORE_THRESH = 0.7
TOKEN_THRESH = 0.08
NUM_CLASSES = 256
ROWS = 5000
BATCH = 16
TILE = 1000
N_T = ROWS // TILE


def _post_kernel(ts_ref, logits_ref, boxes_ref,
                 scores_ref, labels_ref, boxes_out_ref, keep_ref,
                 xywh_ref, pos_ref):
    b = pl.program_id(0)
    r = pl.program_id(1)

    x = logits_ref[0, 0]
    xt = x.T
    m = jnp.max(xt, axis=0, keepdims=True)
    e = jnp.exp(xt - m)
    s = jnp.sum(e, axis=0, keepdims=True)

    st = jnp.concatenate([m, s, m, s, m, s, m, s], axis=0)
    stT = st.T
    m_col = stT[:, 0:1]
    s_col = stT[:, 1:2]

    e_o = jnp.exp(x - m_col)
    pos_ref[0] = (e_o > (TOKEN_THRESH * s_col)).astype(jnp.int8)

    row = lax.broadcasted_iota(jnp.int32, xt.shape, 0)
    valid = row < (NUM_CLASSES - 1)
    em = jnp.where(valid, e, 0.0)
    emax = jnp.max(em, axis=0)
    scores = 1.0 - emax / s[0]
    scores_ref[0, 0, 0] = scores
    keep_ref[0, 0, 0] = (scores > SCORE_THRESH).astype(jnp.int8)

    idx = jnp.where(em == emax[None, :], row, NUM_CLASSES)
    labels_ref[0, 0, 0] = jnp.min(idx, axis=0).astype(jnp.int32)

    @pl.when(r == 0)
    def _():
        bt = boxes_ref[0]
        cxy = bt[0:2]
        half_wh = 0.5 * bt[2:4]
        xyxy = jnp.concatenate([cxy - half_wh, cxy + half_wh], axis=0)
        img_h = ts_ref[b, 0].astype(jnp.float32)
        img_w = ts_ref[b, 1].astype(jnp.float32)
        r4 = lax.broadcasted_iota(jnp.int32, xyxy.shape, 0)
        scale = jnp.where(r4 % 2 == 0, img_w, img_h)
        sb = xyxy * scale
        boxes_out_ref[0] = sb
        xywh_ref[0] = jnp.concatenate([sb[0:2], sb[2:4] - sb[0:2]], axis=0)


@jax.jit
def kernel(pred_logits, pred_boxes, target_sizes):
    grid = (BATCH, N_T)
    ts = target_sizes.astype(jnp.int32)
    boxes_t = jnp.transpose(pred_boxes, (0, 2, 1))
    logits4 = pred_logits.reshape(BATCH, N_T, TILE, NUM_CLASSES)

    out_shapes = (
        jax.ShapeDtypeStruct((BATCH, N_T, 1, TILE), jnp.float32),
        jax.ShapeDtypeStruct((BATCH, N_T, 1, TILE), jnp.int32),
        jax.ShapeDtypeStruct((BATCH, 4, ROWS), jnp.float32),
        jax.ShapeDtypeStruct((BATCH, N_T, 1, TILE), jnp.int8),
        jax.ShapeDtypeStruct((BATCH, 4, ROWS), jnp.float32),
        jax.ShapeDtypeStruct((BATCH, ROWS, NUM_CLASSES), jnp.int8),
    )

    tmap = lambda b, r: (b, r, 0, 0)
    bxmap = lambda b, r: (b, 0, 0)
    pmap = lambda b, r: (b, r, 0)

    scores4, labels4, boxes_f, keep4, xywh_f, pos8 = pl.pallas_call(
        _post_kernel,
        grid=grid,
        in_specs=[
            pl.BlockSpec(memory_space=pltpu.SMEM),
            pl.BlockSpec((1, 1, TILE, NUM_CLASSES), tmap),
            pl.BlockSpec((1, 4, ROWS), bxmap),
        ],
        out_specs=(
            pl.BlockSpec((1, 1, 1, TILE), tmap),
            pl.BlockSpec((1, 1, 1, TILE), tmap),
            pl.BlockSpec((1, 4, ROWS), bxmap),
            pl.BlockSpec((1, 1, 1, TILE), tmap),
            pl.BlockSpec((1, 4, ROWS), bxmap),
            pl.BlockSpec((1, TILE, NUM_CLASSES), pmap),
        ),
        out_shape=out_shapes,
        compiler_params=pltpu.CompilerParams(
            dimension_semantics=("parallel", "arbitrary"),
        ),
    )(ts, logits4, boxes_t)

    scores = scores4.reshape(BATCH, ROWS)
    labels = labels4.reshape(BATCH, ROWS)
    keep = keep4.reshape(BATCH, ROWS).view(jnp.bool_)
    boxes = jnp.transpose(boxes_f, (0, 2, 1))
    xywh = jnp.transpose(xywh_f, (0, 2, 1))
    pos = pos8.view(jnp.bool_)
    return (scores, labels, boxes, keep, xywh, pos)


# speedup vs baseline: 1.3715x; 1.3715x over previous
import jax
import jax.numpy as jnp
from jax import lax
from jax.experimental import pallas as pl
from jax.experimental.pallas import tpu as pltpu

SCORE_THRESH = 0.7
TOKEN_THRESH = 0.08
NUM_CLASSES = 256
ROWS = 5000
BATCH = 16
BOX_R = 40
BOX_C = 500


def _post_kernel(ts_ref, logits_ref, boxes_ref,
                 scores_ref, labels_ref, boxes_out_ref, keep_ref,
                 xywh_ref, pos_ref):
    b = pl.program_id(0)

    x = logits_ref[0]
    xt = x.T
    x2 = xt[:NUM_CLASSES - 1]
    max2 = jnp.max(x2, axis=0)
    mv = jnp.maximum(max2, xt[NUM_CLASSES - 1])
    m = mv[None, :]
    e = jnp.exp(xt - m)
    s = jnp.sum(e, axis=0, keepdims=True)

    st = jnp.concatenate([m, s, m, s, m, s, m, s], axis=0)
    stT = st.T
    m_col = stT[:, 0:1]
    s_col = stT[:, 1:2]

    e_o = jnp.exp(x - m_col)
    pos_ref[0] = (e_o > (TOKEN_THRESH * s_col)).astype(jnp.int8)

    scores = 1.0 - jnp.exp(max2 - mv) / s[0]
    scores_ref[0, 0] = scores
    keep_ref[0, 0] = (scores > SCORE_THRESH).astype(jnp.int8)

    row = lax.broadcasted_iota(jnp.int32, x2.shape, 0)
    idx = jnp.where(x2 == max2[None, :], row, NUM_CLASSES)
    labels_ref[0, 0] = jnp.min(idx, axis=0).astype(jnp.int32)

    bt = boxes_ref[0]
    cxy = bt[0:2]
    half_wh = 0.5 * bt[2:4]
    xyxy = jnp.concatenate([cxy - half_wh, cxy + half_wh], axis=0)
    img_h = ts_ref[b, 0].astype(jnp.float32)
    img_w = ts_ref[b, 1].astype(jnp.float32)
    r4 = lax.broadcasted_iota(jnp.int32, xyxy.shape, 0)
    scale = jnp.where(r4 % 2 == 0, img_w, img_h)
    sb = xyxy * scale
    boxes_out_ref[0] = sb
    xywh_ref[0] = jnp.concatenate([sb[0:2], sb[2:4] - sb[0:2]], axis=0)


@jax.jit
def kernel(pred_logits, pred_boxes, target_sizes):
    grid = (BATCH,)
    ts = target_sizes.astype(jnp.int32)
    boxes_t = jnp.transpose(pred_boxes, (0, 2, 1))

    out_shapes = (
        jax.ShapeDtypeStruct((BATCH, 1, ROWS), jnp.float32),
        jax.ShapeDtypeStruct((BATCH, 1, ROWS), jnp.int32),
        jax.ShapeDtypeStruct((BATCH, 4, ROWS), jnp.float32),
        jax.ShapeDtypeStruct((BATCH, 1, ROWS), jnp.int8),
        jax.ShapeDtypeStruct((BATCH, 4, ROWS), jnp.float32),
        jax.ShapeDtypeStruct((BATCH, ROWS, NUM_CLASSES), jnp.int8),
    )

    bmap = lambda b: (b, 0, 0)

    scores3, labels3, boxes_f, keep3, xywh_f, pos8 = pl.pallas_call(
        _post_kernel,
        grid=grid,
        in_specs=[
            pl.BlockSpec(memory_space=pltpu.SMEM),
            pl.BlockSpec((1, ROWS, NUM_CLASSES), bmap),
            pl.BlockSpec((1, 4, ROWS), bmap),
        ],
        out_specs=(
            pl.BlockSpec((1, 1, ROWS), bmap),
            pl.BlockSpec((1, 1, ROWS), bmap),
            pl.BlockSpec((1, 4, ROWS), bmap),
            pl.BlockSpec((1, 1, ROWS), bmap),
            pl.BlockSpec((1, 4, ROWS), bmap),
            pl.BlockSpec((1, ROWS, NUM_CLASSES), bmap),
        ),
        out_shape=out_shapes,
        compiler_params=pltpu.CompilerParams(
            dimension_semantics=("parallel",),
        ),
    )(ts, pred_logits, boxes_t)

    scores = scores3.reshape(BATCH, ROWS)
    labels = labels3.reshape(BATCH, ROWS)
    keep = keep3.reshape(BATCH, ROWS).view(jnp.bool_)
    boxes = jnp.transpose(boxes_f, (0, 2, 1))
    xywh = jnp.transpose(xywh_f, (0, 2, 1))
    pos = pos8.view(jnp.bool_)
    return (scores, labels, boxes, keep, xywh, pos)
